# baseline (speedup 1.0000x reference)
import jax
import jax.numpy as jnp
from jax import lax
from jax.experimental import pallas as pl
from jax.experimental.pallas import tpu as pltpu
from jax.experimental.pallas import tpu_sc as plsc

_CAP = 50
_MEM = 128
_B = 4096

_NC = 2
_NS = 16
_NW = _NC * _NS
_BPW = _B // _NW
_L = 16
_NG = _BPW // _L
_HB = 80

_NB = 6
_RA = 2


def _sc_body(inputs_hbm, cnt_hbm, mem_hbm, out_hbm, out2_hbm, cnt_out_hbm,
             cnt_v, slot_v, rows_v, hist_v, offs_v, cur_v, list_v,
             buf0, buf1, buf2, buf3, buf4, buf5,
             rd_sem0, rd_sem1, rd_sem2, rd_sem3, rd_sem4, rd_sem5,
             wr_sem0, wr_sem1, wr_sem2, wr_sem3, wr_sem4, wr_sem5,
             w2_sem0, w2_sem1, w2_sem2, w2_sem3, w2_sem4, w2_sem5):
    wid = lax.axis_index("s") * _NC + lax.axis_index("c")
    base = wid * _BPW
    bufs = (buf0, buf1, buf2, buf3, buf4, buf5)
    rd_sems = (rd_sem0, rd_sem1, rd_sem2, rd_sem3, rd_sem4, rd_sem5)
    wr_sems = (wr_sem0, wr_sem1, wr_sem2, wr_sem3, wr_sem4, wr_sem5)
    w2_sems = (w2_sem0, w2_sem1, w2_sem2, w2_sem3, w2_sem4, w2_sem5)

    def rd(s):
        return pltpu.async_copy(
            mem_hbm.at[s, pl.ds(base, _BPW)], bufs[s % _NB],
            rd_sems[s % _NB])

    class _Wr2:
        def __init__(self, s, z):
            self.a = pltpu.async_copy(
                bufs[s % _NB], out_hbm.at[s + z, pl.ds(base, _BPW)],
                wr_sems[s % _NB])
            self.b = pltpu.async_copy(
                bufs[s % _NB], out2_hbm.at[s + z, pl.ds(base, _BPW)],
                w2_sems[s % _NB])

        def wait(self):
            self.a.wait()
            self.b.wait()

    wr = _Wr2

    rds = {s: rd(s) for s in range(_RA)}

    pltpu.sync_copy(cnt_hbm.at[pl.ds(base, _BPW)], cnt_v)
    pltpu.sync_copy(inputs_hbm.at[pl.ds(base, _BPW)], rows_v)
    lanes = lax.iota(jnp.int32, _L)
    zeros = jnp.zeros((_L,), jnp.int32)
    for i in range(_HB // _L):
        hist_v[pl.ds(i * _L, _L)] = zeros
    for i in range(_NG):
        cv = cnt_v[pl.ds(i * _L, _L)]
        slot_v[pl.ds(i * _L, _L)] = lax.rem(cv, _CAP)
        cnt_v[pl.ds(i * _L, _L)] = cv + 1
    pltpu.sync_copy(cnt_v, cnt_out_hbm.at[pl.ds(base, _BPW)])

    for gi in range(_NG):
        svg = slot_v[pl.ds(gi * _L, _L)]
        for l in range(_L):
            s_g = svg[l]
            hv = hist_v[pl.ds(s_g, _L)]
            hist_v[pl.ds(s_g, _L)] = jnp.where(lanes == 0, hv[0] + 1, hv)

    carry = jnp.int32(0)
    for i in range(4):
        h = hist_v[pl.ds(i * _L, _L)]
        c = plsc.cumsum(h)
        excl = c - h + carry
        offs_v[pl.ds(i * _L, _L)] = excl
        cur_v[pl.ds(i * _L, _L)] = excl
        carry = carry + c[_L - 1]

    for gi in range(_NG):
        svg = slot_v[pl.ds(gi * _L, _L)]
        for l in range(_L):
            g = gi * _L + l
            s_g = svg[l]
            cw = cur_v[pl.ds(s_g, _L)]
            p = cw[0]
            cur_v[pl.ds(s_g, _L)] = jnp.where(lanes == 0, p + 1, cw)
            lw = list_v[pl.ds(p, _L)]
            list_v[pl.ds(p, _L)] = jnp.where(lanes == 0, g, lw)

    wrs = {}
    for s in range(_CAP):
        if s + _RA < _CAP:
            if s + _RA >= _NB:
                wrs[s + _RA - _NB].wait()
            rds[s + _RA] = rd(s + _RA)
        rds[s].wait()
        grp = (s // _L) * _L
        st = offs_v[pl.ds(grp, _L)][s % _L]
        n = hist_v[pl.ds(grp, _L)][s % _L]
        buf = bufs[s % _NB]

        def ov_body(t, c, st=st, buf=buf):
            g = list_v[pl.ds(st + t, _L)][0]
            for j in range(_MEM // _L):
                buf[g, pl.ds(j * _L, _L)] = rows_v[g, pl.ds(j * _L, _L)]
            return c

        lax.fori_loop(0, n, ov_body, jnp.int32(0))
        z = plsc.bitcast(buf[0, pl.ds(0, _L)], jnp.int32)[0] & 0
        wrs[s] = wr(s, z)
    for s in range(_CAP - _NB, _CAP):
        wrs[s].wait()


_sc_write = pl.kernel(
    _sc_body,
    out_type=(jax.ShapeDtypeStruct((_CAP, _B, _MEM), jnp.float32),
              jax.ShapeDtypeStruct((_CAP, _B, _MEM), jnp.float32),
              jax.ShapeDtypeStruct((_B,), jnp.int32)),
    mesh=plsc.VectorSubcoreMesh(core_axis_name="c", subcore_axis_name="s",
                                num_cores=_NC, num_subcores=_NS),
    compiler_params=pltpu.CompilerParams(needs_layout_passes=False),
    scratch_types=[
        pltpu.VMEM((_BPW,), jnp.int32),
        pltpu.VMEM((_BPW,), jnp.int32),
        pltpu.VMEM((_BPW, _MEM), jnp.float32),
        pltpu.VMEM((_HB,), jnp.int32),
        pltpu.VMEM((_HB,), jnp.int32),
        pltpu.VMEM((_HB,), jnp.int32),
        pltpu.VMEM((_BPW + _L,), jnp.int32),
    ] + [pltpu.VMEM((_BPW, _MEM), jnp.float32)] * _NB
      + [pltpu.SemaphoreType.DMA] * (3 * _NB),
)


def kernel(inputs, cnt, mem):
    cnt = cnt.astype(jnp.int32)
    mem_t = mem.transpose(1, 0, 2)
    out_t, out2_t, counter = _sc_write(inputs, cnt, mem_t)
    memories = out_t.transpose(1, 0, 2)
    return (memories, counter, out2_t.transpose(1, 0, 2))
